# baseline (speedup 1.0000x reference)
import jax
import jax.numpy as jnp
from jax.experimental import pallas as pl
from jax.experimental.pallas import tpu as pltpu

N = 320000
D = 128
O = 128
B = 64
BLK = 4000
NBLK = N // BLK
NSTREAMS = 4
OUTER = NBLK // NSTREAMS


def _fused_kernel(offs_ref, xa_ref, xb_ref, xc_ref, xd_ref, rows_ref,
                  bounds_ref, sizes_ref,
                  W1_ref, b1_ref, W2_ref, b2_ref, Wp1_ref, bp1_ref, Wp2_ref,
                  bp2_ref, out_ref, acc_ref):
    i = pl.program_id(0)
    total = offs_ref[B - 1]
    nblocks = jax.lax.div(total + (BLK - 1), BLK)

    @pl.when(i == 0)
    def _init():
        acc_ref[:, :] = jnp.zeros_like(acc_ref)

    for which, x_ref in enumerate((xa_ref, xb_ref, xc_ref, xd_ref)):
        blk = NSTREAMS * i + which

        @pl.when(blk < nblocks)
        def _accumulate(x_ref=x_ref, blk=blk):
            x = x_ref[:, :]
            h = jax.lax.dot_general(x, W1_ref[:, :], (((1,), (1,)), ((), ())),
                                    preferred_element_type=jnp.float32)
            h = jnp.maximum(h, -b1_ref[:, :])
            rows = rows_ref[:, :]
            ends = bounds_ref[1:2, :] - blk * BLK
            ltmask = jnp.where(rows < ends, jnp.float32(1.0), jnp.float32(0.0))
            acc_ref[:, :] += jax.lax.dot_general(
                ltmask, h, (((0,), (0,)), ((), ())),
                preferred_element_type=jnp.float32)

    @pl.when(i == OUTER - 1)
    def _finalize():
        accG = acc_ref[:, :]
        seg_iota = jax.lax.broadcasted_iota(jnp.int32, (B, D), 0)
        prevG = jnp.where(seg_iota == 0, jnp.float32(0.0),
                          pltpu.roll(accG, 1, 0))
        summed = (accG - prevG) + sizes_ref[:, :] * b1_ref[:, :]
        pooled = jax.lax.dot_general(
            summed, W2_ref[:, :], (((1,), (1,)), ((), ())),
            preferred_element_type=jnp.float32)
        pooled = pooled + sizes_ref[:, :] * b2_ref[:, :]
        h2 = jax.lax.dot_general(
            pooled, Wp1_ref[:, :], (((1,), (1,)), ((), ())),
            preferred_element_type=jnp.float32)
        h2 = jnp.maximum(h2 + bp1_ref[:, :], 0.0)
        out = jax.lax.dot_general(
            h2, Wp2_ref[:, :], (((1,), (1,)), ((), ())),
            preferred_element_type=jnp.float32)
        out_ref[:, :] = out + bp2_ref[:, :]


def _last_block(offs):
    total = offs[B - 1]
    return jnp.maximum(jax.lax.div(total + (BLK - 1), BLK) - 1, 0)


def _make_x_index_map(which):
    def _map(i, offs):
        return (jnp.minimum(NSTREAMS * i + which, _last_block(offs)), 0)
    return _map


def _const_index_map(i, offs):
    return (0, 0)


@jax.jit
def kernel(batch_num_objects, node_states, W_pre1, b_pre1, W_pre2, b_pre2,
           W_post1, b_post1, W_post2, b_post2):
    sizes = jnp.asarray(batch_num_objects).astype(jnp.int32)
    offsets = jnp.cumsum(sizes)
    starts = offsets - sizes
    bounds = jnp.zeros((8, B), jnp.int32)
    bounds = bounds.at[0, :].set(starts).at[1, :].set(offsets)
    sizes_col = jnp.broadcast_to(
        sizes.astype(jnp.float32)[:, None], (B, D))

    grid_spec = pltpu.PrefetchScalarGridSpec(
        num_scalar_prefetch=1,
        grid=(OUTER,),
        in_specs=[
            pl.BlockSpec((BLK, D), _make_x_index_map(0)),
            pl.BlockSpec((BLK, D), _make_x_index_map(1)),
            pl.BlockSpec((BLK, D), _make_x_index_map(2)),
            pl.BlockSpec((BLK, D), _make_x_index_map(3)),
            pl.BlockSpec((BLK, B), _const_index_map),
            pl.BlockSpec((8, B), _const_index_map),
            pl.BlockSpec((B, D), _const_index_map),
            pl.BlockSpec((D, D), _const_index_map),
            pl.BlockSpec((1, D), _const_index_map),
            pl.BlockSpec((D, D), _const_index_map),
            pl.BlockSpec((1, D), _const_index_map),
            pl.BlockSpec((D, D), _const_index_map),
            pl.BlockSpec((1, D), _const_index_map),
            pl.BlockSpec((O, D), _const_index_map),
            pl.BlockSpec((1, O), _const_index_map),
        ],
        out_specs=pl.BlockSpec((B, O), _const_index_map),
        scratch_shapes=[pltpu.VMEM((B, D), jnp.float32)],
    )

    out_call = pl.pallas_call(
        _fused_kernel,
        grid_spec=grid_spec,
        out_shape=jax.ShapeDtypeStruct((B, O), jnp.float32),
    )
    rows_iota = jax.lax.broadcasted_iota(jnp.int32, (BLK, B), 0)
    out = out_call(offsets, node_states, node_states, node_states,
                   node_states, rows_iota, bounds,
                   sizes_col,
                   W_pre1, b_pre1.reshape(1, D),
                   W_pre2, b_pre2.reshape(1, D),
                   W_post1, b_post1.reshape(1, D),
                   W_post2, b_post2.reshape(1, O))
    return out


# speedup vs baseline: 1.1210x; 1.1210x over previous
import jax
import jax.numpy as jnp
from jax.experimental import pallas as pl
from jax.experimental.pallas import tpu as pltpu

N = 320000
D = 128
O = 128
B = 64
BLK = 8000
NBLK = N // BLK
NSTREAMS = 2
OUTER = NBLK // NSTREAMS


def _fused_kernel(offs_ref, xa_ref, xb_ref, rows_ref,
                  bounds_ref, sizes_ref,
                  W1_ref, b1_ref, W2_ref, b2_ref, Wp1_ref, bp1_ref, Wp2_ref,
                  bp2_ref, out_ref, acc_ref):
    i = pl.program_id(0)
    total = offs_ref[B - 1]
    nblocks = jax.lax.div(total + (BLK - 1), BLK)

    @pl.when(i == 0)
    def _init():
        acc_ref[:, :] = jnp.zeros_like(acc_ref)

    for which, x_ref in enumerate((xa_ref, xb_ref)):
        blk = NSTREAMS * i + which

        @pl.when(blk < nblocks)
        def _accumulate(x_ref=x_ref, blk=blk):
            x = x_ref[:, :]
            h = jax.lax.dot_general(x, W1_ref[:, :], (((1,), (1,)), ((), ())),
                                    preferred_element_type=jnp.float32)
            h = jnp.maximum(h, -b1_ref[:, :])
            rows = rows_ref[:, :]
            ends = bounds_ref[1:2, :] - blk * BLK
            ltmask = jnp.where(rows < ends, jnp.float32(1.0), jnp.float32(0.0))
            acc_ref[:, :] += jax.lax.dot_general(
                ltmask, h, (((0,), (0,)), ((), ())),
                preferred_element_type=jnp.float32)

    @pl.when(i == OUTER - 1)
    def _finalize():
        accG = acc_ref[:, :]
        seg_iota = jax.lax.broadcasted_iota(jnp.int32, (B, D), 0)
        prevG = jnp.where(seg_iota == 0, jnp.float32(0.0),
                          pltpu.roll(accG, 1, 0))
        summed = (accG - prevG) + sizes_ref[:, :] * b1_ref[:, :]
        pooled = jax.lax.dot_general(
            summed, W2_ref[:, :], (((1,), (1,)), ((), ())),
            preferred_element_type=jnp.float32)
        pooled = pooled + sizes_ref[:, :] * b2_ref[:, :]
        h2 = jax.lax.dot_general(
            pooled, Wp1_ref[:, :], (((1,), (1,)), ((), ())),
            preferred_element_type=jnp.float32)
        h2 = jnp.maximum(h2 + bp1_ref[:, :], 0.0)
        out = jax.lax.dot_general(
            h2, Wp2_ref[:, :], (((1,), (1,)), ((), ())),
            preferred_element_type=jnp.float32)
        out_ref[:, :] = out + bp2_ref[:, :]


def _last_block(offs):
    total = offs[B - 1]
    return jnp.maximum(jax.lax.div(total + (BLK - 1), BLK) - 1, 0)


def _make_x_index_map(which):
    def _map(i, offs):
        return (jnp.minimum(NSTREAMS * i + which, _last_block(offs)), 0)
    return _map


def _const_index_map(i, offs):
    return (0, 0)


@jax.jit
def kernel(batch_num_objects, node_states, W_pre1, b_pre1, W_pre2, b_pre2,
           W_post1, b_post1, W_post2, b_post2):
    sizes = jnp.asarray(batch_num_objects).astype(jnp.int32)
    offsets = jnp.cumsum(sizes)
    starts = offsets - sizes
    bounds = jnp.zeros((8, B), jnp.int32)
    bounds = bounds.at[0, :].set(starts).at[1, :].set(offsets)
    sizes_col = jnp.broadcast_to(
        sizes.astype(jnp.float32)[:, None], (B, D))

    grid_spec = pltpu.PrefetchScalarGridSpec(
        num_scalar_prefetch=1,
        grid=(OUTER,),
        in_specs=[
            pl.BlockSpec((BLK, D), _make_x_index_map(0)),
            pl.BlockSpec((BLK, D), _make_x_index_map(1)),
            pl.BlockSpec((BLK, B), _const_index_map),
            pl.BlockSpec((8, B), _const_index_map),
            pl.BlockSpec((B, D), _const_index_map),
            pl.BlockSpec((D, D), _const_index_map),
            pl.BlockSpec((1, D), _const_index_map),
            pl.BlockSpec((D, D), _const_index_map),
            pl.BlockSpec((1, D), _const_index_map),
            pl.BlockSpec((D, D), _const_index_map),
            pl.BlockSpec((1, D), _const_index_map),
            pl.BlockSpec((O, D), _const_index_map),
            pl.BlockSpec((1, O), _const_index_map),
        ],
        out_specs=pl.BlockSpec((B, O), _const_index_map),
        scratch_shapes=[pltpu.VMEM((B, D), jnp.float32)],
    )

    out_call = pl.pallas_call(
        _fused_kernel,
        grid_spec=grid_spec,
        out_shape=jax.ShapeDtypeStruct((B, O), jnp.float32),
    )
    rows_iota = jax.lax.broadcasted_iota(jnp.int32, (BLK, B), 0)
    out = out_call(offsets, node_states, node_states, rows_iota, bounds,
                   sizes_col,
                   W_pre1, b_pre1.reshape(1, D),
                   W_pre2, b_pre2.reshape(1, D),
                   W_post1, b_post1.reshape(1, D),
                   W_post2, b_post2.reshape(1, O))
    return out
